# Initial kernel scaffold; baseline (speedup 1.0000x reference)
import functools

import jax
import jax.numpy as jnp
from jax import lax
from jax.experimental import pallas as pl
from jax.experimental.pallas import tpu as pltpu
from jax.experimental.pallas import tpu_sc as plsc

NC, NS = 2, 16
NW = NC * NS
LANES = 16


def _deg_partials(dst, w, N):
    E = dst.shape[0]
    EW = E // NW
    CH = 400
    NCH = N // CH
    mesh = plsc.VectorSubcoreMesh(core_axis_name="c", subcore_axis_name="s")

    @functools.partial(
        pl.kernel,
        out_type=jax.ShapeDtypeStruct((NC, N), jnp.float32),
        mesh=mesh,
        scratch_types=[
            pltpu.VMEM((EW,), jnp.int32),
            pltpu.VMEM((EW,), jnp.float32),
            pltpu.VMEM((CH,), jnp.float32),
            pltpu.VMEM_SHARED((N,), jnp.float32),
        ],
    )
    def k(dst_hbm, w_hbm, out_hbm, dst_v, w_v, zbuf, deg_sp):
        c = lax.axis_index("c")
        s = lax.axis_index("s")
        wid = c * NS + s

        def zb(i, carry):
            zbuf[pl.ds(i * LANES, LANES)] = jnp.zeros((LANES,), jnp.float32)
            return carry

        lax.fori_loop(0, CH // LANES, zb, 0)
        for j in range((NCH + NS - 1) // NS):
            ch = s + j * NS

            @pl.when(ch < NCH)
            def _():
                pltpu.sync_copy(zbuf, deg_sp.at[pl.ds(ch * CH, CH)])

        plsc.subcore_barrier()
        base = wid * EW
        pltpu.sync_copy(dst_hbm.at[pl.ds(base, EW)], dst_v)
        pltpu.sync_copy(w_hbm.at[pl.ds(base, EW)], w_v)
        pltpu.sync_copy(w_v, deg_sp.at[dst_v], add=True)
        plsc.subcore_barrier()
        for j in range((NCH + NS - 1) // NS):
            ch = s + j * NS

            @pl.when(ch < NCH)
            def _():
                pltpu.sync_copy(deg_sp.at[pl.ds(ch * CH, CH)],
                                out_hbm.at[c, pl.ds(ch * CH, CH)])

    return k(dst, w)


def _aggregate(hp, src, dst, w):
    N, D = hp.shape
    E = src.shape[0]
    EW = E // NW
    B = 400
    NWIN = EW // B
    RT = N // NS
    KD = D // LANES
    mesh = plsc.VectorSubcoreMesh(core_axis_name="c", subcore_axis_name="s")

    @functools.partial(
        pl.kernel,
        out_type=jax.ShapeDtypeStruct((NC, N, D), jnp.float32),
        mesh=mesh,
        scratch_types=[
            pltpu.VMEM((B,), jnp.int32),
            pltpu.VMEM((B,), jnp.int32),
            pltpu.VMEM((B,), jnp.float32),
            pltpu.VMEM((B, D), jnp.float32),
            pltpu.VMEM_SHARED((N, D), jnp.float32),
            pltpu.SemaphoreType.DMA,
        ],
    )
    def k(h_hbm, src_hbm, dst_hbm, w_hbm, out_hbm,
          src_v, dst_v, w_v, rows_v, acc, sem):
        c = lax.axis_index("c")
        s = lax.axis_index("s")
        wid = c * NS + s

        def zr(e, carry):
            for kk in range(KD):
                rows_v[e, pl.ds(kk * LANES, LANES)] = jnp.zeros(
                    (LANES,), jnp.float32)
            return carry

        lax.fori_loop(0, B, zr, 0)
        r0 = s * RT
        pltpu.sync_copy(rows_v, acc.at[pl.ds(r0, B)])
        pltpu.sync_copy(rows_v.at[pl.ds(0, RT - B)],
                        acc.at[pl.ds(r0 + B, RT - B)])
        plsc.subcore_barrier()

        def window(win, carry):
            base = wid * EW + win * B
            pltpu.sync_copy(src_hbm.at[pl.ds(base, B)], src_v)
            pltpu.sync_copy(dst_hbm.at[pl.ds(base, B)], dst_v)
            pltpu.sync_copy(w_hbm.at[pl.ds(base, B)], w_v)
            pltpu.async_copy(h_hbm.at[src_v], rows_v, sem).wait()

            def scale(e, carry2):
                we = w_v[e]
                for kk in range(KD):
                    sl = pl.ds(kk * LANES, LANES)
                    rows_v[e, sl] = rows_v[e, sl] * we
                return carry2

            lax.fori_loop(0, B, scale, 0)
            pltpu.sync_copy(rows_v, acc.at[dst_v], add=True)
            return carry

        lax.fori_loop(0, NWIN, window, 0)
        plsc.subcore_barrier()
        pltpu.sync_copy(acc.at[pl.ds(r0, RT)], out_hbm.at[c, pl.ds(r0, RT)])

    return k(hp, src, dst, w)


def _dis_from_partials(degp):
    N = degp.shape[1]

    def body(d_ref, o_ref):
        deg = d_ref[0, :] + d_ref[1, :] + 1.0
        dis = jnp.where(deg > 0, lax.rsqrt(deg), 0.0)
        o_ref[...] = dis[:, None]

    return pl.pallas_call(
        body, out_shape=jax.ShapeDtypeStruct((N, 1), jnp.float32))(degp)


_RB = 2000


def _first_layer_h(x, W, dis):
    N, D = x.shape

    def body(x_ref, w_ref, d_ref, o_ref):
        o_ref[...] = d_ref[...] * jnp.dot(
            x_ref[...], w_ref[...], preferred_element_type=jnp.float32)

    return pl.pallas_call(
        body,
        grid=(N // _RB,),
        in_specs=[
            pl.BlockSpec((_RB, D), lambda i: (i, 0)),
            pl.BlockSpec((D, D), lambda i: (0, 0)),
            pl.BlockSpec((_RB, 1), lambda i: (i, 0)),
        ],
        out_specs=pl.BlockSpec((_RB, D), lambda i: (i, 0)),
        out_shape=jax.ShapeDtypeStruct((N, D), jnp.float32),
    )(x, W, dis)


def _mid_layer(accp, hp, dis, b, W):
    N, D = hp.shape

    def body(a_ref, h_ref, d_ref, b_ref, w_ref, o_ref):
        ssum = a_ref[0] + a_ref[1] + h_ref[...]
        x2 = jnp.maximum(d_ref[...] * ssum + b_ref[...], 0.0)
        o_ref[...] = d_ref[...] * jnp.dot(
            x2, w_ref[...], preferred_element_type=jnp.float32)

    return pl.pallas_call(
        body,
        grid=(N // _RB,),
        in_specs=[
            pl.BlockSpec((NC, _RB, D), lambda i: (0, i, 0)),
            pl.BlockSpec((_RB, D), lambda i: (i, 0)),
            pl.BlockSpec((_RB, 1), lambda i: (i, 0)),
            pl.BlockSpec((1, D), lambda i: (0, 0)),
            pl.BlockSpec((D, D), lambda i: (0, 0)),
        ],
        out_specs=pl.BlockSpec((_RB, D), lambda i: (i, 0)),
        out_shape=jax.ShapeDtypeStruct((N, D), jnp.float32),
    )(accp, hp, dis, b, W)


def _final_layer(accp, hp, dis, b, Wl, bl):
    N, D = hp.shape

    def body(a_ref, h_ref, d_ref, b_ref, wl_ref, bl_ref, o_ref):
        ssum = a_ref[0] + a_ref[1] + h_ref[...]
        x3 = jnp.maximum(d_ref[...] * ssum + b_ref[...], 0.0)
        o_ref[...] = jnp.dot(
            x3, wl_ref[...], preferred_element_type=jnp.float32) + bl_ref[...]

    return pl.pallas_call(
        body,
        grid=(N // _RB,),
        in_specs=[
            pl.BlockSpec((NC, _RB, D), lambda i: (0, i, 0)),
            pl.BlockSpec((_RB, D), lambda i: (i, 0)),
            pl.BlockSpec((_RB, 1), lambda i: (i, 0)),
            pl.BlockSpec((1, D), lambda i: (0, 0)),
            pl.BlockSpec((D, 1), lambda i: (0, 0)),
            pl.BlockSpec((1, 1), lambda i: (0, 0)),
        ],
        out_specs=pl.BlockSpec((_RB, 1), lambda i: (i, 0)),
        out_shape=jax.ShapeDtypeStruct((N, 1), jnp.float32),
    )(accp, hp, dis, b, Wl, bl)


def kernel(w_x, edge_index, edge_weight, W1, b1, W2, b2, Wl, bl):
    N = w_x.shape[0]
    src = edge_index[0].astype(jnp.int32)
    dst = edge_index[1].astype(jnp.int32)
    w = edge_weight.astype(jnp.float32)

    degp = _deg_partials(dst, w, N)
    dis = _dis_from_partials(degp)
    h1p = _first_layer_h(w_x, W1, dis)
    acc1 = _aggregate(h1p, src, dst, w)
    h2p = _mid_layer(acc1, h1p, dis, b1.reshape(1, -1), W2)
    acc2 = _aggregate(h2p, src, dst, w)
    return _final_layer(acc2, h2p, dis, b2.reshape(1, -1), Wl,
                        bl.reshape(1, 1))


# speedup vs baseline: 16.4496x; 16.4496x over previous
import functools

import jax
import jax.numpy as jnp
from jax import lax
from jax.experimental import pallas as pl
from jax.experimental.pallas import tpu as pltpu
from jax.experimental.pallas import tpu_sc as plsc

NC, NS = 2, 16
NW = NC * NS
LANES = 16


def _deg_partials(dst, w, N):
    E = dst.shape[0]
    EW = E // NW
    CH = 400
    NCH = N // CH
    mesh = plsc.VectorSubcoreMesh(core_axis_name="c", subcore_axis_name="s",
                                  num_cores=NC, num_subcores=NS)

    @functools.partial(
        pl.kernel,
        out_type=jax.ShapeDtypeStruct((NC * N,), jnp.float32),
        mesh=mesh,
        scratch_types=[
            pltpu.VMEM((EW,), jnp.int32),
            pltpu.VMEM((EW,), jnp.float32),
            pltpu.VMEM((CH,), jnp.float32),
            pltpu.VMEM_SHARED((N,), jnp.float32),
        ],
    )
    def k(dst_hbm, w_hbm, out_hbm, dst_v, w_v, zbuf, deg_sp):
        c = lax.axis_index("c")
        s = lax.axis_index("s")
        wid = c * NS + s

        def zb(i, carry):
            zbuf[pl.ds(i * LANES, LANES)] = jnp.zeros((LANES,), jnp.float32)
            return carry

        lax.fori_loop(0, CH // LANES, zb, 0)
        for j in range((NCH + NS - 1) // NS):
            ch = s + j * NS

            @pl.when(ch < NCH)
            def _():
                pltpu.sync_copy(zbuf, deg_sp.at[pl.ds(ch * CH, CH)])

        plsc.subcore_barrier()
        base = wid * EW
        pltpu.sync_copy(dst_hbm.at[pl.ds(base, EW)], dst_v)
        pltpu.sync_copy(w_hbm.at[pl.ds(base, EW)], w_v)
        pltpu.sync_copy(w_v, deg_sp.at[dst_v], add=True)
        plsc.subcore_barrier()
        for j in range((NCH + NS - 1) // NS):
            ch = s + j * NS

            @pl.when(ch < NCH)
            def _():
                pltpu.sync_copy(deg_sp.at[pl.ds(ch * CH, CH)], zbuf)
                pltpu.sync_copy(zbuf, out_hbm.at[pl.ds(c * N + ch * CH, CH)])

    return k(dst, w).reshape(NC, N)


def _aggregate(hp, src, dst, w):
    N, D = hp.shape
    E = src.shape[0]
    EW = E // NW
    B = 200
    NWIN = EW // B
    NRCH = N // B
    KD = D // LANES
    mesh = plsc.VectorSubcoreMesh(core_axis_name="c", subcore_axis_name="s",
                                  num_cores=NC, num_subcores=NS)

    @functools.partial(
        pl.kernel,
        out_type=jax.ShapeDtypeStruct((NC, N, D), jnp.float32),
        mesh=mesh,
        scratch_types=[
            pltpu.VMEM((B,), jnp.int32),
            pltpu.VMEM((B,), jnp.int32),
            pltpu.VMEM((B,), jnp.float32),
            pltpu.VMEM((B, D), jnp.float32),
            pltpu.VMEM_SHARED((N, D), jnp.float32),
            pltpu.SemaphoreType.DMA,
        ],
    )
    def k(h_hbm, src_hbm, dst_hbm, w_hbm, out_hbm,
          src_v, dst_v, w_v, rows_v, acc, sem):
        c = lax.axis_index("c")
        s = lax.axis_index("s")
        wid = c * NS + s

        def zr(e, carry):
            for kk in range(KD):
                rows_v[e, pl.ds(kk * LANES, LANES)] = jnp.zeros(
                    (LANES,), jnp.float32)
            return carry

        lax.fori_loop(0, B, zr, 0)
        for j in range((NRCH + NS - 1) // NS):
            ch = s + j * NS

            @pl.when(ch < NRCH)
            def _():
                pltpu.sync_copy(rows_v, acc.at[pl.ds(ch * B, B)])

        plsc.subcore_barrier()

        def window(win, carry):
            base = wid * EW + win * B
            pltpu.sync_copy(src_hbm.at[pl.ds(base, B)], src_v)
            pltpu.sync_copy(dst_hbm.at[pl.ds(base, B)], dst_v)
            pltpu.sync_copy(w_hbm.at[pl.ds(base, B)], w_v)
            pltpu.async_copy(h_hbm.at[src_v], rows_v, sem).wait()

            def scale(g, carry2):
                w16 = w_v[pl.ds(g * LANES, LANES)]
                for j in range(LANES):
                    we = w16[j]
                    e = g * LANES + j
                    for kk in range(KD):
                        sl = pl.ds(kk * LANES, LANES)
                        rows_v[e, sl] = rows_v[e, sl] * we
                return carry2

            lax.fori_loop(0, B // LANES, scale, 0)
            if B % LANES:
                w16t = w_v[pl.ds(B - LANES, LANES)]
                for j in range(LANES - (B % LANES), LANES):
                    we = w16t[j]
                    e = B - LANES + j
                    for kk in range(KD):
                        sl = pl.ds(kk * LANES, LANES)
                        rows_v[e, sl] = rows_v[e, sl] * we
            pltpu.sync_copy(rows_v, acc.at[dst_v], add=True)
            return carry

        lax.fori_loop(0, NWIN, window, 0)
        plsc.subcore_barrier()
        for j in range((NRCH + NS - 1) // NS):
            ch = s + j * NS

            @pl.when(ch < NRCH)
            def _():
                pltpu.sync_copy(acc.at[pl.ds(ch * B, B)], rows_v)
                pltpu.sync_copy(rows_v, out_hbm.at[c, pl.ds(ch * B, B)])

    return k(hp, src, dst, w)


def _dis_from_partials(degp):
    N = degp.shape[1]

    def body(d_ref, o_ref):
        deg = d_ref[0, :] + d_ref[1, :] + 1.0
        dis = jnp.where(deg > 0, lax.rsqrt(deg), 0.0)
        o_ref[...] = dis[:, None]

    return pl.pallas_call(
        body, out_shape=jax.ShapeDtypeStruct((N, 1), jnp.float32))(degp)


_RB = 2000


def _first_layer_h(x, W, dis):
    N, D = x.shape

    def body(x_ref, w_ref, d_ref, o_ref):
        o_ref[...] = d_ref[...] * jnp.dot(
            x_ref[...], w_ref[...], preferred_element_type=jnp.float32)

    return pl.pallas_call(
        body,
        grid=(N // _RB,),
        in_specs=[
            pl.BlockSpec((_RB, D), lambda i: (i, 0)),
            pl.BlockSpec((D, D), lambda i: (0, 0)),
            pl.BlockSpec((_RB, 1), lambda i: (i, 0)),
        ],
        out_specs=pl.BlockSpec((_RB, D), lambda i: (i, 0)),
        out_shape=jax.ShapeDtypeStruct((N, D), jnp.float32),
    )(x, W, dis)


def _mid_layer(accp, hp, dis, b, W):
    N, D = hp.shape

    def body(a_ref, h_ref, d_ref, b_ref, w_ref, o_ref):
        ssum = a_ref[0] + a_ref[1] + h_ref[...]
        x2 = jnp.maximum(d_ref[...] * ssum + b_ref[...], 0.0)
        o_ref[...] = d_ref[...] * jnp.dot(
            x2, w_ref[...], preferred_element_type=jnp.float32)

    return pl.pallas_call(
        body,
        grid=(N // _RB,),
        in_specs=[
            pl.BlockSpec((NC, _RB, D), lambda i: (0, i, 0)),
            pl.BlockSpec((_RB, D), lambda i: (i, 0)),
            pl.BlockSpec((_RB, 1), lambda i: (i, 0)),
            pl.BlockSpec((1, D), lambda i: (0, 0)),
            pl.BlockSpec((D, D), lambda i: (0, 0)),
        ],
        out_specs=pl.BlockSpec((_RB, D), lambda i: (i, 0)),
        out_shape=jax.ShapeDtypeStruct((N, D), jnp.float32),
    )(accp, hp, dis, b, W)


def _final_layer(accp, hp, dis, b, Wl, bl):
    N, D = hp.shape

    def body(a_ref, h_ref, d_ref, b_ref, wl_ref, bl_ref, o_ref):
        ssum = a_ref[0] + a_ref[1] + h_ref[...]
        x3 = jnp.maximum(d_ref[...] * ssum + b_ref[...], 0.0)
        o_ref[...] = jnp.dot(
            x3, wl_ref[...], preferred_element_type=jnp.float32) + bl_ref[...]

    return pl.pallas_call(
        body,
        grid=(N // _RB,),
        in_specs=[
            pl.BlockSpec((NC, _RB, D), lambda i: (0, i, 0)),
            pl.BlockSpec((_RB, D), lambda i: (i, 0)),
            pl.BlockSpec((_RB, 1), lambda i: (i, 0)),
            pl.BlockSpec((1, D), lambda i: (0, 0)),
            pl.BlockSpec((D, 1), lambda i: (0, 0)),
            pl.BlockSpec((1, 1), lambda i: (0, 0)),
        ],
        out_specs=pl.BlockSpec((_RB, 1), lambda i: (i, 0)),
        out_shape=jax.ShapeDtypeStruct((N, 1), jnp.float32),
    )(accp, hp, dis, b, Wl, bl)


def kernel(w_x, edge_index, edge_weight, W1, b1, W2, b2, Wl, bl):
    N = w_x.shape[0]
    src = edge_index[0].astype(jnp.int32)
    dst = edge_index[1].astype(jnp.int32)
    w = edge_weight.astype(jnp.float32)

    degp = _deg_partials(dst, w, N)
    dis = _dis_from_partials(degp)
    h1p = _first_layer_h(w_x, W1, dis)
    acc1 = _aggregate(h1p, src, dst, w)
    h2p = _mid_layer(acc1, h1p, dis, b1.reshape(1, -1), W2)
    acc2 = _aggregate(h2p, src, dst, w)
    return _final_layer(acc2, h2p, dis, b2.reshape(1, -1), Wl,
                        bl.reshape(1, 1))
